# baseline (speedup 1.0000x reference)
import functools

import jax
import jax.numpy as jnp
from jax.experimental import pallas as pl
from jax.experimental.pallas import tpu as pltpu

K = 16
RB = 256
N_LAYERS = 2


def _bf(a):
    return a.astype(jnp.bfloat16).astype(jnp.float32)


def _layer_body(xT_ref, xrT_ref, z_ref, Wc_ref, bc_ref, W1r_ref, b1_ref,
                v_ref, W2_ref, b2_ref, w3_ref, b3_ref,
                xo_ref, xoT_ref, res_ref, *, rb_size, n, k):
    rb = pl.program_id(1)

    xt = xT_ref[0]
    xrT = xrT_ref[0]
    v = v_ref[...].astype(jnp.float32)
    W2 = W2_ref[...]
    b2 = b2_ref[...]
    w3 = w3_ref[...].astype(jnp.float32)
    b3 = b3_ref[0:1, 0:1]

    ctx = jnp.dot(z_ref[0], Wc_ref[...],
                  preferred_element_type=jnp.float32) + bc_ref[...]
    u = jnp.dot(ctx.astype(jnp.bfloat16), W1r_ref[...],
                preferred_element_type=jnp.float32) + b1_ref[...]

    sq_row = jnp.sum(xt * xt, axis=0, keepdims=True)
    sq_col = jnp.sum(xrT * xrT, axis=1, keepdims=True)
    xtb = xt.astype(jnp.bfloat16).astype(jnp.float32)
    xrb = xrT.astype(jnp.bfloat16).astype(jnp.float32)
    dot = jnp.zeros((rb_size, n), dtype=jnp.float32)
    for d in range(3):
        dot = dot + xrb[:, d:d + 1] * xtb[d:d + 1, :]
    d2 = sq_col + sq_row - 2.0 * dot

    lane_i = jax.lax.broadcasted_iota(jnp.int32, (rb_size, n), 1)
    lane_f = lane_i.astype(jnp.float32)
    row_g = jax.lax.broadcasted_iota(jnp.int32, (rb_size, n), 0) + rb * rb_size
    d2 = jnp.where(lane_i == row_g, d2 + 1000000.0, d2)

    xi = [xrT[:, d:d + 1] for d in range(3)]

    d2x = jnp.zeros((rb_size, n), dtype=jnp.float32)
    for d in range(3):
        rr = (xt[d:d + 1, :] - xi[d]) + 1e-12
        d2x = d2x + rr * rr

    accw = jnp.zeros((rb_size, n), dtype=jnp.float32)
    sumw = jnp.zeros((rb_size, 1), dtype=jnp.float32)
    denom = jnp.zeros((rb_size, 1), dtype=jnp.float32)
    nf = jnp.float32(n)
    for _ in range(k):
        m = jnp.min(d2, axis=1, keepdims=True)
        cand = jnp.where(d2 == m, lane_f, nf)
        imin = jnp.min(cand, axis=1, keepdims=True)
        sel = (lane_f == imin).astype(jnp.float32)
        d2 = d2 + sel * 1.0e9
        dsel = jnp.sum(d2x * sel, axis=1, keepdims=True)
        dij = jnp.sqrt(dsel)

        h1 = jnp.maximum(_bf(dij) * v + u, 0.0)
        h2 = jnp.maximum(
            jnp.dot(h1.astype(jnp.bfloat16), W2,
                    preferred_element_type=jnp.float32) + b2, 0.0
        )
        wr = jnp.sum(_bf(h2) * w3, axis=1, keepdims=True) + b3
        w = jnp.tanh(wr)
        denom = denom + jnp.abs(w)
        sumw = sumw + w
        accw = accw + sel * w

    inv = 1.0 / (denom + 1e-08)
    delta = []
    for d in range(3):
        s = jnp.sum(accw * xt[d:d + 1, :], axis=1, keepdims=True)
        delta.append((s - sumw * xi[d]) * inv)

    xon = jnp.concatenate(
        [xi[d] + delta[d] for d in range(3)], axis=1
    )
    xo_ref[0] = xon
    xoT_ref[0] = xon.T

    dsq = delta[0] ** 2 + delta[1] ** 2 + delta[2] ** 2
    r = jnp.sum(dsq)
    m8 = jax.lax.broadcasted_iota(jnp.int32, (8, 128), 0)
    m128 = jax.lax.broadcasted_iota(jnp.int32, (8, 128), 1)
    res_ref[0, 0] = jnp.where((m8 == 0) & (m128 == 0), r, 0.0)


def _epilogue_body(xN_ref, p1_ref, p2_ref, xc_ref, rn_ref):
    xb = xN_ref[0]
    mean = jnp.mean(xb, axis=0, keepdims=True)
    xc_ref[0] = xb - mean
    total = jnp.sum(p1_ref[0]) + jnp.sum(p2_ref[0])
    rn = jnp.sqrt(total + 1e-12)
    rn_ref[0] = jnp.full((8, 128), rn, dtype=jnp.float32)


def kernel(x, z, Wc, bc, W1, b1, W2, b2, W3, b3):
    B, N, _ = x.shape
    E = W2.shape[0]
    D = Wc.shape[0]
    H = Wc.shape[1]
    NB = N // RB

    zb = z.reshape(B, 1, D).astype(jnp.bfloat16)
    Wcb = Wc.astype(jnp.bfloat16)
    bc1 = bc.reshape(1, -1)
    W1rb = W1[1:, :].astype(jnp.bfloat16)
    b1row = b1.reshape(1, -1)
    v = W1[0:1, :].astype(jnp.bfloat16)
    W2b = W2.astype(jnp.bfloat16)
    w3row = W3.reshape(1, E).astype(jnp.bfloat16)
    b3row = jnp.broadcast_to(b3.reshape(1, 1), (8, 128))
    b2row = b2.reshape(1, -1)

    layer = pl.pallas_call(
        functools.partial(_layer_body, rb_size=RB, n=N, k=K),
        grid=(B, NB),
        compiler_params=pltpu.CompilerParams(
            dimension_semantics=("parallel", "parallel")),
        in_specs=[
            pl.BlockSpec((1, 3, N), lambda b, rb: (b, 0, 0)),
            pl.BlockSpec((1, RB, 3), lambda b, rb: (b, rb, 0)),
            pl.BlockSpec((1, 1, D), lambda b, rb: (b, 0, 0)),
            pl.BlockSpec((D, H), lambda b, rb: (0, 0)),
            pl.BlockSpec((1, H), lambda b, rb: (0, 0)),
            pl.BlockSpec((H, E), lambda b, rb: (0, 0)),
            pl.BlockSpec((1, E), lambda b, rb: (0, 0)),
            pl.BlockSpec((1, E), lambda b, rb: (0, 0)),
            pl.BlockSpec((E, E), lambda b, rb: (0, 0)),
            pl.BlockSpec((1, E), lambda b, rb: (0, 0)),
            pl.BlockSpec((1, E), lambda b, rb: (0, 0)),
            pl.BlockSpec((8, 128), lambda b, rb: (0, 0)),
        ],
        out_specs=[
            pl.BlockSpec((1, RB, 3), lambda b, rb: (b, rb, 0)),
            pl.BlockSpec((1, 3, RB), lambda b, rb: (b, 0, rb)),
            pl.BlockSpec((1, 1, 8, 128), lambda b, rb: (b, rb, 0, 0)),
        ],
        out_shape=[
            jax.ShapeDtypeStruct((B, N, 3), jnp.float32),
            jax.ShapeDtypeStruct((B, 3, N), jnp.float32),
            jax.ShapeDtypeStruct((B, NB, 8, 128), jnp.float32),
        ],
    )

    xN = x
    xT = jnp.transpose(x, (0, 2, 1))
    partials = []
    for _ in range(N_LAYERS):
        xN, xT, p = layer(xT, xN, zb, Wcb, bc1, W1rb, b1row,
                          v, W2b, b2row, w3row, b3row)
        partials.append(p)

    xc, rn = pl.pallas_call(
        _epilogue_body,
        grid=(B,),
        in_specs=[
            pl.BlockSpec((1, N, 3), lambda b: (b, 0, 0)),
            pl.BlockSpec((1, NB, 8, 128), lambda b: (b, 0, 0, 0)),
            pl.BlockSpec((1, NB, 8, 128), lambda b: (b, 0, 0, 0)),
        ],
        out_specs=[
            pl.BlockSpec((1, N, 3), lambda b: (b, 0, 0)),
            pl.BlockSpec((1, 8, 128), lambda b: (b, 0, 0)),
        ],
        out_shape=[
            jax.ShapeDtypeStruct((B, N, 3), jnp.float32),
            jax.ShapeDtypeStruct((B, 8, 128), jnp.float32),
        ],
    )(xN, partials[0], partials[1])

    return xc, rn[:, 0, 0]


# speedup vs baseline: 1.0310x; 1.0310x over previous
import functools

import jax
import jax.numpy as jnp
from jax.experimental import pallas as pl
from jax.experimental.pallas import tpu as pltpu

K = 16
RB = 256
N_LAYERS = 2


def _bf(a):
    return a.astype(jnp.bfloat16).astype(jnp.float32)


def _layer_body(xT_ref, xrT_ref, z_ref, Wc_ref, bc_ref, W1r_ref, b1_ref,
                v_ref, W2_ref, b2_ref, w3_ref, b3_ref,
                xo_ref, xoT_ref, res_ref, *, rb_size, n, k):
    rb = pl.program_id(1)

    xt = xT_ref[0]
    xrT = xrT_ref[0]
    v = v_ref[...].astype(jnp.float32)
    W2 = W2_ref[...]
    b2 = b2_ref[...]
    w3 = w3_ref[...].astype(jnp.float32)
    b3 = b3_ref[0:1, 0:1]

    ctx = jnp.dot(z_ref[0], Wc_ref[...],
                  preferred_element_type=jnp.float32) + bc_ref[...]
    u = jnp.dot(ctx.astype(jnp.bfloat16), W1r_ref[...],
                preferred_element_type=jnp.float32) + b1_ref[...]

    sq_row = jnp.sum(xt * xt, axis=0, keepdims=True)
    sq_col = jnp.sum(xrT * xrT, axis=1, keepdims=True)
    xtb = xt.astype(jnp.bfloat16).astype(jnp.float32)
    xrb = xrT.astype(jnp.bfloat16).astype(jnp.float32)
    dot = jnp.zeros((rb_size, n), dtype=jnp.float32)
    for d in range(3):
        dot = dot + xrb[:, d:d + 1] * xtb[d:d + 1, :]
    d2 = sq_col + sq_row - 2.0 * dot

    lane_i = jax.lax.broadcasted_iota(jnp.int32, (rb_size, n), 1)
    lane_f = lane_i.astype(jnp.float32)
    row_g = jax.lax.broadcasted_iota(jnp.int32, (rb_size, n), 0) + rb * rb_size
    d2 = jnp.where(lane_i == row_g, d2 + 1000000.0, d2)

    xi = [xrT[:, d:d + 1] for d in range(3)]

    d2x = jnp.zeros((rb_size, n), dtype=jnp.float32)
    for d in range(3):
        rr = (xt[d:d + 1, :] - xi[d]) + 1e-12
        d2x = d2x + rr * rr

    accw = jnp.zeros((rb_size, n), dtype=jnp.float32)
    sumw = jnp.zeros((rb_size, 1), dtype=jnp.float32)
    denom = jnp.zeros((rb_size, 1), dtype=jnp.float32)
    nf = jnp.float32(n)
    for _ in range(k):
        m = jnp.min(d2, axis=1, keepdims=True)
        cand = jnp.where(d2 == m, lane_f, nf)
        imin = jnp.min(cand, axis=1, keepdims=True)
        sel = lane_f == imin
        d2 = jnp.where(sel, 1.0e9, d2)
        dsel = jnp.sum(jnp.where(sel, d2x, 0.0), axis=1, keepdims=True)
        dij = jnp.sqrt(dsel)

        h1 = jnp.maximum(_bf(dij) * v + u, 0.0)
        h2 = jnp.maximum(
            jnp.dot(h1.astype(jnp.bfloat16), W2,
                    preferred_element_type=jnp.float32) + b2, 0.0
        )
        wr = jnp.sum(_bf(h2) * w3, axis=1, keepdims=True) + b3
        w = jnp.tanh(wr)
        denom = denom + jnp.abs(w)
        sumw = sumw + w
        accw = accw + jnp.where(sel, w, 0.0)

    inv = 1.0 / (denom + 1e-08)
    delta = []
    for d in range(3):
        s = jnp.sum(accw * xt[d:d + 1, :], axis=1, keepdims=True)
        delta.append((s - sumw * xi[d]) * inv)

    xon = jnp.concatenate(
        [xi[d] + delta[d] for d in range(3)], axis=1
    )
    xo_ref[0] = xon
    xoT_ref[0] = xon.T

    dsq = delta[0] ** 2 + delta[1] ** 2 + delta[2] ** 2
    r = jnp.sum(dsq)
    m8 = jax.lax.broadcasted_iota(jnp.int32, (8, 128), 0)
    m128 = jax.lax.broadcasted_iota(jnp.int32, (8, 128), 1)
    res_ref[0, 0] = jnp.where((m8 == 0) & (m128 == 0), r, 0.0)


def _epilogue_body(xN_ref, p1_ref, p2_ref, xc_ref, rn_ref):
    xb = xN_ref[0]
    mean = jnp.mean(xb, axis=0, keepdims=True)
    xc_ref[0] = xb - mean
    total = jnp.sum(p1_ref[0]) + jnp.sum(p2_ref[0])
    rn = jnp.sqrt(total + 1e-12)
    rn_ref[0] = jnp.full((8, 128), rn, dtype=jnp.float32)


def kernel(x, z, Wc, bc, W1, b1, W2, b2, W3, b3):
    B, N, _ = x.shape
    E = W2.shape[0]
    D = Wc.shape[0]
    H = Wc.shape[1]
    NB = N // RB

    zb = z.reshape(B, 1, D).astype(jnp.bfloat16)
    Wcb = Wc.astype(jnp.bfloat16)
    bc1 = bc.reshape(1, -1)
    W1rb = W1[1:, :].astype(jnp.bfloat16)
    b1row = b1.reshape(1, -1)
    v = W1[0:1, :].astype(jnp.bfloat16)
    W2b = W2.astype(jnp.bfloat16)
    w3row = W3.reshape(1, E).astype(jnp.bfloat16)
    b3row = jnp.broadcast_to(b3.reshape(1, 1), (8, 128))
    b2row = b2.reshape(1, -1)

    layer = pl.pallas_call(
        functools.partial(_layer_body, rb_size=RB, n=N, k=K),
        grid=(B, NB),
        compiler_params=pltpu.CompilerParams(
            dimension_semantics=("parallel", "parallel")),
        in_specs=[
            pl.BlockSpec((1, 3, N), lambda b, rb: (b, 0, 0)),
            pl.BlockSpec((1, RB, 3), lambda b, rb: (b, rb, 0)),
            pl.BlockSpec((1, 1, D), lambda b, rb: (b, 0, 0)),
            pl.BlockSpec((D, H), lambda b, rb: (0, 0)),
            pl.BlockSpec((1, H), lambda b, rb: (0, 0)),
            pl.BlockSpec((H, E), lambda b, rb: (0, 0)),
            pl.BlockSpec((1, E), lambda b, rb: (0, 0)),
            pl.BlockSpec((1, E), lambda b, rb: (0, 0)),
            pl.BlockSpec((E, E), lambda b, rb: (0, 0)),
            pl.BlockSpec((1, E), lambda b, rb: (0, 0)),
            pl.BlockSpec((1, E), lambda b, rb: (0, 0)),
            pl.BlockSpec((8, 128), lambda b, rb: (0, 0)),
        ],
        out_specs=[
            pl.BlockSpec((1, RB, 3), lambda b, rb: (b, rb, 0)),
            pl.BlockSpec((1, 3, RB), lambda b, rb: (b, 0, rb)),
            pl.BlockSpec((1, 1, 8, 128), lambda b, rb: (b, rb, 0, 0)),
        ],
        out_shape=[
            jax.ShapeDtypeStruct((B, N, 3), jnp.float32),
            jax.ShapeDtypeStruct((B, 3, N), jnp.float32),
            jax.ShapeDtypeStruct((B, NB, 8, 128), jnp.float32),
        ],
    )

    xN = x
    xT = jnp.transpose(x, (0, 2, 1))
    partials = []
    for _ in range(N_LAYERS):
        xN, xT, p = layer(xT, xN, zb, Wcb, bc1, W1rb, b1row,
                          v, W2b, b2row, w3row, b3row)
        partials.append(p)

    xc, rn = pl.pallas_call(
        _epilogue_body,
        grid=(B,),
        in_specs=[
            pl.BlockSpec((1, N, 3), lambda b: (b, 0, 0)),
            pl.BlockSpec((1, NB, 8, 128), lambda b: (b, 0, 0, 0)),
            pl.BlockSpec((1, NB, 8, 128), lambda b: (b, 0, 0, 0)),
        ],
        out_specs=[
            pl.BlockSpec((1, N, 3), lambda b: (b, 0, 0)),
            pl.BlockSpec((1, 8, 128), lambda b: (b, 0, 0)),
        ],
        out_shape=[
            jax.ShapeDtypeStruct((B, N, 3), jnp.float32),
            jax.ShapeDtypeStruct((B, 8, 128), jnp.float32),
        ],
    )(xN, partials[0], partials[1])

    return xc, rn[:, 0, 0]
